# baseline (speedup 1.0000x reference)
import functools

import jax
import jax.numpy as jnp
from jax import lax
from jax.experimental import pallas as pl
from jax.experimental.pallas import tpu as pltpu
from jax.experimental.pallas import tpu_sc as plsc

F = 128
F2 = 64
N_PER_SET = 50000
NC = 2
NS = 16
NB = 128
NB2 = 26
PER_H = NB * NB2
HALF = PER_H * NS * NC
PAD = HALF - 2 * N_PER_SET
IDX_LEN = 2 * HALF

BLK = 2000


def _sc_gather_body(table_hbm, idx_hbm, out_hbm, idx_v, bufs, gsems, wsems):
  wid = lax.axis_index("s") * NC + lax.axis_index("c")
  base = wid * PER_H
  pltpu.sync_copy(idx_hbm.at[pl.ds(base, PER_H)], idx_v.at[pl.ds(0, PER_H)])
  pltpu.sync_copy(idx_hbm.at[pl.ds(HALF + base, PER_H)],
                  idx_v.at[pl.ds(PER_H, PER_H)])

  def gath(off, t, b):
    return pltpu.make_async_copy(
        table_hbm.at[idx_v.at[pl.ds(off + t * NB, NB)]], bufs[b], gsems[b])

  def wout(half, t, b):
    return pltpu.make_async_copy(
        bufs[b], out_hbm.at[half, pl.ds(base + t * NB, NB)], wsems[b])

  gath(0, 0, 0).start()
  gath(PER_H, 0, 1).start()

  def step(k, carry):
    gath(0, k, 0).wait()
    wout(0, k, 0).start()
    wout(0, k, 0).wait()

    @pl.when(k + 1 < NB2)
    def _():
      gath(0, k + 1, 0).start()

    gath(PER_H, k, 1).wait()
    wout(1, k, 1).start()
    wout(1, k, 1).wait()

    @pl.when(k + 1 < NB2)
    def _():
      gath(PER_H, k + 1, 1).start()

    return carry

  lax.fori_loop(0, NB2, step, 0)


@functools.lru_cache(maxsize=None)
def _make_sc_gather():
  return pl.kernel(
      _sc_gather_body,
      out_type=jax.ShapeDtypeStruct((2, HALF, F2), jnp.float32),
      mesh=plsc.VectorSubcoreMesh(core_axis_name="c", subcore_axis_name="s"),
      compiler_params=pltpu.CompilerParams(use_tc_tiling_on_sc=False),
      scratch_types=[
          pltpu.VMEM((2 * PER_H,), jnp.int32),
          [pltpu.VMEM((NB, F2), jnp.float32)] * 2,
          [pltpu.SemaphoreType.DMA] * 2,
          [pltpu.SemaphoreType.DMA] * 2,
      ],
  )


def _mlp_probs(xi_ref, xj_ref, w5_ref, b5_ref, w6_ref, b6_ref):
  p = xi_ref[0].astype(jnp.float32) * xj_ref[0].astype(jnp.float32)
  h = lax.dot_general(
      p, w5_ref[...], (((1,), (1,)), ((), ())),
      preferred_element_type=jnp.float32)
  h = jnp.maximum(h + b5_ref[...], 0.0)
  logits = lax.dot_general(
      h, w6_ref[...], (((1,), (1,)), ((), ())),
      preferred_element_type=jnp.float32) + b6_ref[...]
  m = jnp.max(logits, axis=-1, keepdims=True)
  e = jnp.exp(logits - m)
  return e / jnp.sum(e, axis=-1, keepdims=True)


def _mlp_body_r(xi_ref, xj_ref, w5_ref, b5_ref, w6_ref, b6_ref, out_ref):
  out_ref[...] = _mlp_probs(xi_ref, xj_ref, w5_ref, b5_ref, w6_ref, b6_ref)


def _mlp_body_s(xi_ref, xj_ref, w5_ref, b5_ref, w6_ref, b6_ref, out_ref):
  probs = _mlp_probs(xi_ref, xj_ref, w5_ref, b5_ref, w6_ref, b6_ref)
  out_ref[...] = probs[:, None, :]


def _mlp_call(body, gathered, weights, block_off, out_shape, out_spec):
  grid = N_PER_SET // BLK
  return pl.pallas_call(
      body,
      grid=(grid,),
      in_specs=[
          pl.BlockSpec((1, BLK, F), lambda i: (0, i + block_off, 0)),
          pl.BlockSpec((1, BLK, F), lambda i: (1, i + block_off, 0)),
          pl.BlockSpec((F, F), lambda i: (0, 0)),
          pl.BlockSpec((1, F), lambda i: (0, 0)),
          pl.BlockSpec((2, F), lambda i: (0, 0)),
          pl.BlockSpec((1, 2), lambda i: (0, 0)),
      ],
      out_specs=out_spec,
      out_shape=out_shape,
  )(gathered, gathered, *weights)


def kernel(x, cliques_r, cliques_s, node_features, lin5_w, lin5_b, lin6_w,
           lin6_b):
  del x
  zpad = jnp.zeros((PAD,), jnp.int32)
  idx_all = jnp.concatenate([
      cliques_r[:, 0], cliques_s[:, 0], zpad,
      cliques_r[:, 1], cliques_s[:, 1], zpad,
  ])
  tpack = lax.bitcast_convert_type(
      node_features.astype(jnp.bfloat16).reshape(-1, F2, 2), jnp.float32)
  gathered = _make_sc_gather()(tpack, idx_all)
  g16 = lax.bitcast_convert_type(gathered, jnp.bfloat16).reshape(2, HALF, F)

  weights = (lin5_w, lin5_b.reshape(1, F), lin6_w, lin6_b.reshape(1, 2))
  edge_probs_r = _mlp_call(
      _mlp_body_r, g16, weights, 0,
      jax.ShapeDtypeStruct((N_PER_SET, 2), jnp.float32),
      pl.BlockSpec((BLK, 2), lambda i: (i, 0)))
  edge_probs_s = _mlp_call(
      _mlp_body_s, g16, weights, N_PER_SET // BLK,
      jax.ShapeDtypeStruct((N_PER_SET, 1, 2), jnp.float32),
      pl.BlockSpec((BLK, 1, 2), lambda i: (i, 0, 0)))
  return (edge_probs_r, edge_probs_s)


# speedup vs baseline: 1.6894x; 1.6894x over previous
import functools

import jax
import jax.numpy as jnp
from jax import lax
from jax.experimental import pallas as pl
from jax.experimental.pallas import tpu as pltpu
from jax.experimental.pallas import tpu_sc as plsc

F = 128
N_PER_SET = 50000
NC = 2
NS = 16
NB = 128
NB2 = 26
PER_H = NB * NB2
HALF = PER_H * NS * NC
PAD = HALF - 2 * N_PER_SET
IDX_LEN = 2 * HALF

BLK = 2000


def _sc_gather_body(table_hbm, idx_hbm, out_hbm, idx_v, bufs, gsems, wsems):
  wid = lax.axis_index("s") * NC + lax.axis_index("c")
  base = wid * PER_H
  pltpu.sync_copy(idx_hbm.at[pl.ds(base, PER_H)], idx_v.at[pl.ds(0, PER_H)])
  pltpu.sync_copy(idx_hbm.at[pl.ds(HALF + base, PER_H)],
                  idx_v.at[pl.ds(PER_H, PER_H)])

  def gath(off, t, b):
    return pltpu.make_async_copy(
        table_hbm.at[idx_v.at[pl.ds(off + t * NB, NB)]], bufs[b], gsems[b])

  def wout(half, t, b):
    return pltpu.make_async_copy(
        bufs[b], out_hbm.at[half, pl.ds(base + t * NB, NB)], wsems[b])

  gath(0, 0, 0).start()
  gath(PER_H, 0, 1).start()

  def step(k, carry):
    gath(0, k, 0).wait()
    wout(0, k, 0).start()
    wout(0, k, 0).wait()

    @pl.when(k + 1 < NB2)
    def _():
      gath(0, k + 1, 0).start()

    gath(PER_H, k, 1).wait()
    wout(1, k, 1).start()
    wout(1, k, 1).wait()

    @pl.when(k + 1 < NB2)
    def _():
      gath(PER_H, k + 1, 1).start()

    return carry

  lax.fori_loop(0, NB2, step, 0)


@functools.lru_cache(maxsize=None)
def _make_sc_gather():
  return pl.kernel(
      _sc_gather_body,
      out_type=jax.ShapeDtypeStruct((2, HALF, F), jnp.float32),
      mesh=plsc.VectorSubcoreMesh(core_axis_name="c", subcore_axis_name="s"),
      scratch_types=[
          pltpu.VMEM((2 * PER_H,), jnp.int32),
          [pltpu.VMEM((NB, F), jnp.float32)] * 2,
          [pltpu.SemaphoreType.DMA] * 2,
          [pltpu.SemaphoreType.DMA] * 2,
      ],
  )


def _mlp_probs(xi_ref, xj_ref, w5_ref, b5_ref, w6_ref, b6_ref):
  p = xi_ref[0] * xj_ref[0]
  h = lax.dot_general(
      p, w5_ref[...], (((1,), (1,)), ((), ())),
      preferred_element_type=jnp.float32)
  h = jnp.maximum(h + b5_ref[...], 0.0)
  logits = lax.dot_general(
      h, w6_ref[...], (((1,), (1,)), ((), ())),
      preferred_element_type=jnp.float32) + b6_ref[...]
  m = jnp.max(logits, axis=-1, keepdims=True)
  e = jnp.exp(logits - m)
  return e / jnp.sum(e, axis=-1, keepdims=True)


def _mlp_body_r(xi_ref, xj_ref, w5_ref, b5_ref, w6_ref, b6_ref, out_ref):
  out_ref[...] = _mlp_probs(xi_ref, xj_ref, w5_ref, b5_ref, w6_ref, b6_ref)


def _mlp_body_s(xi_ref, xj_ref, w5_ref, b5_ref, w6_ref, b6_ref, out_ref):
  probs = _mlp_probs(xi_ref, xj_ref, w5_ref, b5_ref, w6_ref, b6_ref)
  out_ref[...] = probs[:, None, :]


def _mlp_call(body, gathered, weights, block_off, out_shape, out_spec):
  grid = N_PER_SET // BLK
  return pl.pallas_call(
      body,
      grid=(grid,),
      in_specs=[
          pl.BlockSpec((1, BLK, F), lambda i: (0, i + block_off, 0)),
          pl.BlockSpec((1, BLK, F), lambda i: (1, i + block_off, 0)),
          pl.BlockSpec((F, F), lambda i: (0, 0)),
          pl.BlockSpec((1, F), lambda i: (0, 0)),
          pl.BlockSpec((2, F), lambda i: (0, 0)),
          pl.BlockSpec((1, 2), lambda i: (0, 0)),
      ],
      out_specs=out_spec,
      out_shape=out_shape,
  )(gathered, gathered, *weights)


def kernel(x, cliques_r, cliques_s, node_features, lin5_w, lin5_b, lin6_w,
           lin6_b):
  del x
  zpad = jnp.zeros((PAD,), jnp.int32)
  idx_all = jnp.concatenate([
      cliques_r[:, 0], cliques_s[:, 0], zpad,
      cliques_r[:, 1], cliques_s[:, 1], zpad,
  ])
  gathered = _make_sc_gather()(node_features, idx_all)

  weights = (lin5_w, lin5_b.reshape(1, F), lin6_w, lin6_b.reshape(1, 2))
  edge_probs_r = _mlp_call(
      _mlp_body_r, gathered, weights, 0,
      jax.ShapeDtypeStruct((N_PER_SET, 2), jnp.float32),
      pl.BlockSpec((BLK, 2), lambda i: (i, 0)))
  edge_probs_s = _mlp_call(
      _mlp_body_s, gathered, weights, N_PER_SET // BLK,
      jax.ShapeDtypeStruct((N_PER_SET, 1, 2), jnp.float32),
      pl.BlockSpec((BLK, 1, 2), lambda i: (i, 0, 0)))
  return (edge_probs_r, edge_probs_s)


# speedup vs baseline: 1.6923x; 1.0017x over previous
import functools

import jax
import jax.numpy as jnp
from jax import lax
from jax.experimental import pallas as pl
from jax.experimental.pallas import tpu as pltpu
from jax.experimental.pallas import tpu_sc as plsc

F = 128
N_PER_SET = 50000
NC = 2
NS = 16
NB = 128
NB2 = 26
PER_H = NB * NB2
HALF = PER_H * NS * NC
PAD = HALF - 2 * N_PER_SET
IDX_LEN = 2 * HALF

BLK = 2000


def _sc_gather_body(table_hbm, idx_hbm, out_hbm, idx_v, bufs, gsems, wsems):
  wid = lax.axis_index("s") * NC + lax.axis_index("c")
  base = wid * PER_H
  pltpu.sync_copy(idx_hbm.at[pl.ds(base, PER_H)], idx_v.at[pl.ds(0, PER_H)])
  pltpu.sync_copy(idx_hbm.at[pl.ds(HALF + base, PER_H)],
                  idx_v.at[pl.ds(PER_H, PER_H)])

  def gath(off, t, b):
    return pltpu.make_async_copy(
        table_hbm.at[idx_v.at[pl.ds(off + t * NB, NB)]], bufs[b], gsems[b])

  def wout(half, t, b):
    return pltpu.make_async_copy(
        bufs[b], out_hbm.at[half, pl.ds(base + t * NB, NB)], wsems[b])

  gath(0, 0, 0).start()
  gath(PER_H, 0, 1).start()

  def step(k, carry):
    gath(0, k, 0).wait()
    wout(0, k, 0).start()
    wout(0, k, 0).wait()

    @pl.when(k + 1 < NB2)
    def _():
      gath(0, k + 1, 0).start()

    gath(PER_H, k, 1).wait()
    wout(1, k, 1).start()
    wout(1, k, 1).wait()

    @pl.when(k + 1 < NB2)
    def _():
      gath(PER_H, k + 1, 1).start()

    return carry

  lax.fori_loop(0, NB2, step, 0)


@functools.lru_cache(maxsize=None)
def _make_sc_gather():
  return pl.kernel(
      _sc_gather_body,
      out_type=jax.ShapeDtypeStruct((2, HALF, F), jnp.float32),
      mesh=plsc.VectorSubcoreMesh(core_axis_name="c", subcore_axis_name="s"),
      scratch_types=[
          pltpu.VMEM((2 * PER_H,), jnp.int32),
          [pltpu.VMEM((NB, F), jnp.float32)] * 2,
          [pltpu.SemaphoreType.DMA] * 2,
          [pltpu.SemaphoreType.DMA] * 2,
      ],
  )


def _mlp_probs(g_ref, w5_ref, b5_ref, w6_ref, b6_ref):
  p = g_ref[0] * g_ref[1]
  h = lax.dot_general(
      p, w5_ref[...], (((1,), (1,)), ((), ())),
      preferred_element_type=jnp.float32)
  h = jnp.maximum(h + b5_ref[...], 0.0)
  logits = lax.dot_general(
      h, w6_ref[...], (((1,), (1,)), ((), ())),
      preferred_element_type=jnp.float32) + b6_ref[...]
  m = jnp.max(logits, axis=-1, keepdims=True)
  e = jnp.exp(logits - m)
  return e / jnp.sum(e, axis=-1, keepdims=True)


def _mlp_body_r(g_ref, w5_ref, b5_ref, w6_ref, b6_ref, out_ref):
  out_ref[...] = _mlp_probs(g_ref, w5_ref, b5_ref, w6_ref, b6_ref)


def _mlp_body_s(g_ref, w5_ref, b5_ref, w6_ref, b6_ref, out_ref):
  probs = _mlp_probs(g_ref, w5_ref, b5_ref, w6_ref, b6_ref)
  out_ref[...] = probs[:, None, :]


def _mlp_call(body, gathered, weights, block_off, out_shape, out_spec):
  grid = N_PER_SET // BLK
  return pl.pallas_call(
      body,
      grid=(grid,),
      in_specs=[
          pl.BlockSpec((2, BLK, F), lambda i: (0, i + block_off, 0)),
          pl.BlockSpec((F, F), lambda i: (0, 0)),
          pl.BlockSpec((1, F), lambda i: (0, 0)),
          pl.BlockSpec((2, F), lambda i: (0, 0)),
          pl.BlockSpec((1, 2), lambda i: (0, 0)),
      ],
      out_specs=out_spec,
      out_shape=out_shape,
  )(gathered, *weights)


def kernel(x, cliques_r, cliques_s, node_features, lin5_w, lin5_b, lin6_w,
           lin6_b):
  del x
  zpad = jnp.zeros((PAD,), jnp.int32)
  idx_all = jnp.concatenate([
      cliques_r[:, 0], cliques_s[:, 0], zpad,
      cliques_r[:, 1], cliques_s[:, 1], zpad,
  ])
  gathered = _make_sc_gather()(node_features, idx_all)

  weights = (lin5_w, lin5_b.reshape(1, F), lin6_w, lin6_b.reshape(1, 2))
  edge_probs_r = _mlp_call(
      _mlp_body_r, gathered, weights, 0,
      jax.ShapeDtypeStruct((N_PER_SET, 2), jnp.float32),
      pl.BlockSpec((BLK, 2), lambda i: (i, 0)))
  edge_probs_s = _mlp_call(
      _mlp_body_s, gathered, weights, N_PER_SET // BLK,
      jax.ShapeDtypeStruct((N_PER_SET, 1, 2), jnp.float32),
      pl.BlockSpec((BLK, 1, 2), lambda i: (i, 0, 0)))
  return (edge_probs_r, edge_probs_s)


# speedup vs baseline: 1.7755x; 1.0492x over previous
import functools

import jax
import jax.numpy as jnp
from jax import lax
from jax.experimental import pallas as pl
from jax.experimental.pallas import tpu as pltpu
from jax.experimental.pallas import tpu_sc as plsc

F = 128
N_PER_SET = 50000
NC = 2
NS = 16
NB = 128
NB2 = 26
PER_H = NB * NB2
HALF = PER_H * NS * NC
PAD = HALF - 2 * N_PER_SET
IDX_LEN = 2 * HALF

BLK = 2000


def _sc_gather_body(table_hbm, idx_hbm, out_hbm, idx_v, bufs, gsems, wsems):
  wid = lax.axis_index("s") * NC + lax.axis_index("c")
  base = wid * PER_H
  pltpu.sync_copy(idx_hbm.at[pl.ds(base, PER_H)], idx_v.at[pl.ds(0, PER_H)])
  pltpu.sync_copy(idx_hbm.at[pl.ds(HALF + base, PER_H)],
                  idx_v.at[pl.ds(PER_H, PER_H)])

  def gath(off, t, b):
    return pltpu.make_async_copy(
        table_hbm.at[idx_v.at[pl.ds(off + t * NB, NB)]], bufs[b], gsems[b])

  def wout(half, t, b):
    return pltpu.make_async_copy(
        bufs[b], out_hbm.at[half, pl.ds(base + t * NB, NB)], wsems[b])

  gath(0, 0, 0).start()
  gath(PER_H, 0, 1).start()

  def step(k, carry):
    gath(0, k, 0).wait()
    wout(0, k, 0).start()
    wout(0, k, 0).wait()

    @pl.when(k + 1 < NB2)
    def _():
      gath(0, k + 1, 0).start()

    gath(PER_H, k, 1).wait()
    wout(1, k, 1).start()
    wout(1, k, 1).wait()

    @pl.when(k + 1 < NB2)
    def _():
      gath(PER_H, k + 1, 1).start()

    return carry

  lax.fori_loop(0, NB2, step, 0)


@functools.lru_cache(maxsize=None)
def _make_sc_gather():
  return pl.kernel(
      _sc_gather_body,
      out_type=jax.ShapeDtypeStruct((2, HALF, F), jnp.float32),
      mesh=plsc.VectorSubcoreMesh(core_axis_name="c", subcore_axis_name="s"),
      scratch_types=[
          pltpu.VMEM((2 * PER_H,), jnp.int32),
          [pltpu.VMEM((NB, F), jnp.float32)] * 2,
          [pltpu.SemaphoreType.DMA] * 2,
          [pltpu.SemaphoreType.DMA] * 2,
      ],
  )


def _mlp_probs(g_ref, w5_ref, b5_ref, w6_ref, b6_ref):
  p = g_ref[0] * g_ref[1]
  h = lax.dot_general(
      p, w5_ref[...], (((1,), (1,)), ((), ())),
      preferred_element_type=jnp.float32)
  h = jnp.maximum(h + b5_ref[...], 0.0)
  logits = lax.dot_general(
      h, w6_ref[...], (((1,), (1,)), ((), ())),
      preferred_element_type=jnp.float32) + b6_ref[...]
  m = jnp.max(logits, axis=-1, keepdims=True)
  e = jnp.exp(logits - m)
  return e / jnp.sum(e, axis=-1, keepdims=True)


def _mlp_body_r(g_ref, w5_ref, b5_ref, w6_ref, b6_ref, out_ref):
  out_ref[...] = _mlp_probs(g_ref, w5_ref, b5_ref, w6_ref, b6_ref)


def _mlp_body_s(g_ref, w5_ref, b5_ref, w6_ref, b6_ref, out_ref):
  out_ref[...] = _mlp_probs(g_ref, w5_ref, b5_ref, w6_ref, b6_ref)


def _mlp_call(body, gathered, weights, block_off, out_shape, out_spec):
  grid = N_PER_SET // BLK
  return pl.pallas_call(
      body,
      grid=(grid,),
      in_specs=[
          pl.BlockSpec((2, BLK, F), lambda i: (0, i + block_off, 0)),
          pl.BlockSpec((F, F), lambda i: (0, 0)),
          pl.BlockSpec((1, F), lambda i: (0, 0)),
          pl.BlockSpec((2, F), lambda i: (0, 0)),
          pl.BlockSpec((1, 2), lambda i: (0, 0)),
      ],
      out_specs=out_spec,
      out_shape=out_shape,
  )(gathered, *weights)


def kernel(x, cliques_r, cliques_s, node_features, lin5_w, lin5_b, lin6_w,
           lin6_b):
  del x
  zpad = jnp.zeros((PAD,), jnp.int32)
  idx_all = jnp.concatenate([
      cliques_r[:, 0], cliques_s[:, 0], zpad,
      cliques_r[:, 1], cliques_s[:, 1], zpad,
  ])
  gathered = _make_sc_gather()(node_features, idx_all)

  weights = (lin5_w, lin5_b.reshape(1, F), lin6_w, lin6_b.reshape(1, 2))
  edge_probs_r = _mlp_call(
      _mlp_body_r, gathered, weights, 0,
      jax.ShapeDtypeStruct((N_PER_SET, 2), jnp.float32),
      pl.BlockSpec((BLK, 2), lambda i: (i, 0)))
  edge_probs_s = _mlp_call(
      _mlp_body_s, gathered, weights, N_PER_SET // BLK,
      jax.ShapeDtypeStruct((N_PER_SET, 2), jnp.float32),
      pl.BlockSpec((BLK, 2), lambda i: (i, 0)))
  return (edge_probs_r, edge_probs_s[:, None, :])
